# baseline (speedup 1.0000x reference)
import functools
import math

import jax
import jax.numpy as jnp
from jax import lax
from jax.experimental import pallas as pl
from jax.experimental.pallas import tpu as pltpu
from jax.experimental.pallas import tpu_sc as plsc

N, E, B, L, DIN, D = 10000, 320000, 256, 1000, 78, 128
NPAD = 10240
NC, NS = 2, 16
CH = 64
EPAD = 327680
NCHUNK = EPAD // CH
NITER = NCHUNK // (NC * NS)
GS = 32
BR = 1024
TN = 32
LD = L * D


def _seg_partials(h, src2d, dst2d, zeros):
    mesh = plsc.VectorSubcoreMesh(core_axis_name="c", subcore_axis_name="s")
    NBUF = 4

    @functools.partial(
        pl.kernel,
        out_type=jax.ShapeDtypeStruct((NC * NPAD, D), jnp.float32),
        mesh=mesh,
        scratch_types=[
            pltpu.VMEM((GS, CH), jnp.int32),
            pltpu.VMEM((GS, CH), jnp.int32),
        ] + [pltpu.VMEM((CH, D), jnp.float32) for _ in range(NBUF)] + [
            pltpu.VMEM_SHARED((NPAD, D), jnp.float32),
        ] + [pltpu.SemaphoreType.DMA for _ in range(NBUF)],
    )
    def k(h_hbm, src_hbm, dst_hbm, zeros_hbm, out_hbm,
          sidx, didx, *rest):
        rows = rest[:NBUF]
        accum = rest[NBUF]
        sems = rest[NBUF + 1:]
        c = lax.axis_index("c")
        s = lax.axis_index("s")
        rps = NPAD // NS
        pltpu.sync_copy(zeros_hbm.at[pl.ds(s * rps, rps)],
                        accum.at[pl.ds(s * rps, rps)])
        cb = (c * NS + s) * NITER
        plsc.subcore_barrier()

        def group(g, carry):
            pltpu.sync_copy(src_hbm.at[pl.ds(cb + g * GS, GS)], sidx)
            pltpu.sync_copy(dst_hbm.at[pl.ds(cb + g * GS, GS)], didx)
            for j in range(NBUF):
                pltpu.async_copy(h_hbm.at[sidx.at[j]], rows[j], sems[j])

            def body(t, carry2):
                for j in range(NBUF):
                    i = NBUF * t + j
                    pltpu.make_async_copy(
                        h_hbm.at[sidx.at[i]], rows[j], sems[j]).wait()
                    pltpu.sync_copy(rows[j], accum.at[didx.at[i]], add=True)

                    @pl.when(i + NBUF < GS)
                    def _():
                        pltpu.async_copy(
                            h_hbm.at[sidx.at[i + NBUF]], rows[j], sems[j])
                return carry2

            lax.fori_loop(0, GS // NBUF, body, 0)
            return carry

        lax.fori_loop(0, NITER // GS, group, 0)
        plsc.subcore_barrier()
        pltpu.sync_copy(accum.at[pl.ds(s * rps, rps)],
                        out_hbm.at[pl.ds(c * NPAD + s * rps, rps)])

    out = k(h, src2d, dst2d, zeros)
    return out.reshape(NC, NPAD, D)


def _conv_mlp(h, p0, p1, w1t, b1, w2t, b2):
    nb = NPAD // BR

    def body(h_ref, p0_ref, p1_ref, w1_ref, b1_ref, w2_ref, b2_ref, z_ref, st_ref):
        i = pl.program_id(0)

        @pl.when(i == 0)
        def _():
            st_ref[...] = jnp.zeros_like(st_ref)

        x = h_ref[...] + p0_ref[...] + p1_ref[...]
        z1 = jnp.maximum(
            jnp.dot(x, w1_ref[...], preferred_element_type=jnp.float32)
            + b1_ref[...], 0.0)
        z = jnp.maximum(
            jnp.dot(z1, w2_ref[...], preferred_element_type=jnp.float32)
            + b2_ref[...], 0.0)
        rid = i * BR + lax.broadcasted_iota(jnp.int32, (BR, 1), 0)
        z = jnp.where(rid < N, z, 0.0)
        z_ref[...] = z
        st_ref[0:1, :] += jnp.sum(z, axis=0, keepdims=True)
        st_ref[1:2, :] += jnp.sum(z * z, axis=0, keepdims=True)

    return pl.pallas_call(
        body,
        grid=(nb,),
        in_specs=[
            pl.BlockSpec((BR, D), lambda i: (i, 0)),
            pl.BlockSpec((BR, D), lambda i: (i, 0)),
            pl.BlockSpec((BR, D), lambda i: (i, 0)),
            pl.BlockSpec((D, D), lambda i: (0, 0)),
            pl.BlockSpec((1, D), lambda i: (0, 0)),
            pl.BlockSpec((D, D), lambda i: (0, 0)),
            pl.BlockSpec((1, D), lambda i: (0, 0)),
        ],
        out_specs=[
            pl.BlockSpec((BR, D), lambda i: (i, 0)),
            pl.BlockSpec((8, D), lambda i: (0, 0)),
        ],
        out_shape=[
            jax.ShapeDtypeStruct((NPAD, D), jnp.float32),
            jax.ShapeDtypeStruct((8, D), jnp.float32),
        ],
    )(h, p0, p1, w1t, b1, w2t, b2)


def _bn_apply(z, st, g, bb):
    nb = NPAD // BR

    def body(z_ref, st_ref, g_ref, b_ref, h_ref):
        mu = st_ref[0:1, :] / N
        var = st_ref[1:2, :] / N - mu * mu
        a = g_ref[...] * lax.rsqrt(var + 1e-5)
        c = b_ref[...] - mu * a
        h_ref[...] = z_ref[...] * a + c

    return pl.pallas_call(
        body,
        grid=(nb,),
        in_specs=[
            pl.BlockSpec((BR, D), lambda i: (i, 0)),
            pl.BlockSpec((8, D), lambda i: (0, 0)),
            pl.BlockSpec((1, D), lambda i: (0, 0)),
            pl.BlockSpec((1, D), lambda i: (0, 0)),
        ],
        out_specs=pl.BlockSpec((BR, D), lambda i: (i, 0)),
        out_shape=jax.ShapeDtypeStruct((NPAD, D), jnp.float32),
    )(z, st, g, bb)


def _attention(h, batch_pad, target3, table_pad):
    scale = 1.0 / math.sqrt(float(D))

    def body(batch_ref, tgt_ref, table_ref, h_ref, pooled_ref):
        b = pl.program_id(0)
        bat = batch_ref[...]
        off0 = jnp.sum((bat < b).astype(jnp.int32))
        off1 = jnp.sum((bat <= b).astype(jnp.int32))
        tgt_row = tgt_ref[0]
        oht = (lax.broadcasted_iota(jnp.int32, (32, L), 0) == tgt_row
               ).astype(jnp.float32)
        emb_b = lax.dot_general(oht, table_ref[...],
                                (((0,), (0,)), ((), ())),
                                preferred_element_type=jnp.float32)
        nblk = (off1 - off0 + TN - 1) // TN

        def blk(j, acc):
            start = off0 + j * TN
            rows = h_ref[pl.ds(start, TN), :]
            s = lax.dot_general(rows, emb_b, (((1,), (1,)), ((), ())),
                                preferred_element_type=jnp.float32) * scale
            m = jnp.max(s, axis=1, keepdims=True)
            e = jnp.exp(s - m)
            p = e / jnp.sum(e, axis=1, keepdims=True)
            o = jnp.dot(p, emb_b, preferred_element_type=jnp.float32)
            rid = start + lax.broadcasted_iota(jnp.int32, (TN, 1), 0)
            o = jnp.where(rid < off1, o, 0.0)
            return acc + jnp.sum(o, axis=0, keepdims=True)

        pooled_ref[0] = lax.fori_loop(
            0, nblk, blk, jnp.zeros((1, D), jnp.float32))

    return pl.pallas_call(
        body,
        grid=(B,),
        in_specs=[
            pl.BlockSpec((NPAD // D, D), lambda b: (0, 0)),
            pl.BlockSpec((1, 1, L), lambda b: (b, 0, 0)),
            pl.BlockSpec((32, D), lambda b: (0, 0)),
            pl.BlockSpec((NPAD, D), lambda b: (0, 0)),
        ],
        out_specs=pl.BlockSpec((1, 1, D), lambda b: (b, 0, 0)),
        out_shape=jax.ShapeDtypeStruct((B, 1, D), jnp.float32),
    )(batch_pad, target3, table_pad, h)


def _xt_matmul(targett, table_pad, w, bias):
    kl = 8
    nk = L // kl

    def body(t_ref, table_ref, w_ref, b_ref, o_ref):
        k = pl.program_id(0)

        @pl.when(k == 0)
        def _():
            o_ref[...] = jnp.broadcast_to(b_ref[...], (B, D))

        acc = jnp.zeros((B, D), jnp.float32)
        for j in range(kl):
            qb = lax.dot_general(table_ref[...], w_ref[:, j * D:(j + 1) * D],
                                 (((1,), (1,)), ((), ())),
                                 preferred_element_type=jnp.float32)
            oht = (lax.broadcasted_iota(jnp.int32, (32, B), 0)
                   == t_ref[j:j + 1, :]).astype(jnp.float32)
            acc += lax.dot_general(oht, qb, (((0,), (0,)), ((), ())),
                                   preferred_element_type=jnp.float32)
        o_ref[...] += acc

    return pl.pallas_call(
        body,
        grid=(nk,),
        in_specs=[
            pl.BlockSpec((kl, B), lambda k: (k, 0)),
            pl.BlockSpec((32, D), lambda k: (0, 0)),
            pl.BlockSpec((D, kl * D), lambda k: (0, k)),
            pl.BlockSpec((1, D), lambda k: (0, 0)),
        ],
        out_specs=pl.BlockSpec((B, D), lambda k: (0, 0)),
        out_shape=jax.ShapeDtypeStruct((B, D), jnp.float32),
    )(targett, table_pad, w, bias)


def _head(pooled, xt, fxd_t, fxd_b, fc1_t, fc1_b, fc2_t, fc2_b, outw_p, outb):
    def body(p_ref, xt_ref, w0_ref, b0_ref, w1_ref, b1_ref, w2_ref, b2_ref,
             w3_ref, b3_ref, o_ref):
        xg = jnp.maximum(
            jnp.dot(p_ref[...], w0_ref[...], preferred_element_type=jnp.float32)
            + b0_ref[...], 0.0)
        xc = jnp.concatenate([xg, xt_ref[...]], axis=1)
        h1 = jnp.maximum(
            jnp.dot(xc, w1_ref[...], preferred_element_type=jnp.float32)
            + b1_ref[...], 0.0)
        h2 = jnp.maximum(
            jnp.dot(h1, w2_ref[...], preferred_element_type=jnp.float32)
            + b2_ref[...], 0.0)
        o_ref[...] = (jnp.dot(h2, w3_ref[...], preferred_element_type=jnp.float32)
                      + b3_ref[...])

    return pl.pallas_call(
        body,
        out_shape=jax.ShapeDtypeStruct((B, D), jnp.float32),
    )(pooled, xt, fxd_t, fxd_b, fc1_t, fc1_b, fc2_t, fc2_b, outw_p, outb)


def kernel(x, edge_index, batch, target, params):
    pad_dst = N + jnp.arange(EPAD - E, dtype=jnp.int32) % (NPAD - N)
    pad_src = jnp.arange(EPAD - E, dtype=jnp.int32) % N
    src2d = jnp.concatenate(
        [edge_index[0], pad_src]).reshape(NCHUNK, CH)
    dst2d = jnp.concatenate(
        [edge_index[1], pad_dst]).reshape(NCHUNK, CH)
    zeros = jnp.zeros((NPAD, D), jnp.float32)
    table_pad = jnp.pad(params['emb'], ((0, 32 - 26), (0, 0)))
    xt = _xt_matmul(target.T, table_pad, params['fc1xt_W'],
                    params['fc1xt_b'].reshape(1, D))
    h = jnp.pad(x, ((0, NPAD - N), (0, D - DIN)))
    for c in (1, 2, 3):
        w1 = params['conv%d_W1' % c]
        if c == 1:
            w1 = jnp.pad(w1, ((0, 0), (0, D - DIN)))
        parts = _seg_partials(h, src2d, dst2d, zeros)
        z, st = _conv_mlp(h, parts[0], parts[1], w1.T,
                          params['conv%d_b1' % c].reshape(1, D),
                          params['conv%d_W2' % c].T,
                          params['conv%d_b2' % c].reshape(1, D))
        h = _bn_apply(z, st, params['bn%d_g' % c].reshape(1, D),
                      params['bn%d_b' % c].reshape(1, D))

    batch_pad = jnp.concatenate(
        [batch, jnp.full((NPAD - N,), B, jnp.int32)]).reshape(NPAD // D, D)
    pooled = _attention(h, batch_pad, target.reshape(B, 1, L), table_pad)
    pooled = pooled.reshape(B, D)

    outw_p = jnp.pad(params['out_W'].T, ((0, 0), (0, D - 1)))
    out = _head(pooled, xt,
                params['fc1xd_W'].T, params['fc1xd_b'].reshape(1, D),
                params['fc1_W'].T, params['fc1_b'].reshape(1, 1024),
                params['fc2_W'].T, params['fc2_b'].reshape(1, 256),
                outw_p, jnp.pad(params['out_b'].reshape(1, 1),
                                ((0, 0), (0, D - 1))))
    return out[:, :1]


# speedup vs baseline: 1.1134x; 1.1134x over previous
import functools
import math

import jax
import jax.numpy as jnp
from jax import lax
from jax.experimental import pallas as pl
from jax.experimental.pallas import tpu as pltpu
from jax.experimental.pallas import tpu_sc as plsc

N, E, B, L, DIN, D = 10000, 320000, 256, 1000, 78, 128
NPAD = 10240
NC, NS = 2, 16
CH = 64
EPAD = 327680
NCHUNK = EPAD // CH
NITER = NCHUNK // (NC * NS)
GS = 32
BR = 1024
TN = 128
LD = L * D


def _seg_partials(h, src2d, dst2d, zeros):
    mesh = plsc.VectorSubcoreMesh(core_axis_name="c", subcore_axis_name="s")
    NBUF = 4

    @functools.partial(
        pl.kernel,
        out_type=jax.ShapeDtypeStruct((NC * NPAD, D), jnp.float32),
        mesh=mesh,
        scratch_types=[
            pltpu.VMEM((GS, CH), jnp.int32),
            pltpu.VMEM((GS, CH), jnp.int32),
        ] + [pltpu.VMEM((CH, D), jnp.float32) for _ in range(NBUF)] + [
            pltpu.VMEM_SHARED((NPAD, D), jnp.float32),
        ] + [pltpu.SemaphoreType.DMA for _ in range(NBUF)],
    )
    def k(h_hbm, src_hbm, dst_hbm, zeros_hbm, out_hbm,
          sidx, didx, *rest):
        rows = rest[:NBUF]
        accum = rest[NBUF]
        sems = rest[NBUF + 1:]
        c = lax.axis_index("c")
        s = lax.axis_index("s")
        rps = NPAD // NS
        pltpu.sync_copy(zeros_hbm.at[pl.ds(s * rps, rps)],
                        accum.at[pl.ds(s * rps, rps)])
        cb = (c * NS + s) * NITER
        plsc.subcore_barrier()

        def group(g, carry):
            pltpu.sync_copy(src_hbm.at[pl.ds(cb + g * GS, GS)], sidx)
            pltpu.sync_copy(dst_hbm.at[pl.ds(cb + g * GS, GS)], didx)
            for j in range(NBUF):
                pltpu.async_copy(h_hbm.at[sidx.at[j]], rows[j], sems[j])

            def body(t, carry2):
                for j in range(NBUF):
                    i = NBUF * t + j
                    pltpu.make_async_copy(
                        h_hbm.at[sidx.at[i]], rows[j], sems[j]).wait()
                    pltpu.sync_copy(rows[j], accum.at[didx.at[i]], add=True)

                    @pl.when(i + NBUF < GS)
                    def _():
                        pltpu.async_copy(
                            h_hbm.at[sidx.at[i + NBUF]], rows[j], sems[j])
                return carry2

            lax.fori_loop(0, GS // NBUF, body, 0)
            return carry

        lax.fori_loop(0, NITER // GS, group, 0)
        plsc.subcore_barrier()
        pltpu.sync_copy(accum.at[pl.ds(s * rps, rps)],
                        out_hbm.at[pl.ds(c * NPAD + s * rps, rps)])

    out = k(h, src2d, dst2d, zeros)
    return out.reshape(NC, NPAD, D)


def _conv_mlp(h, p0, p1, w1t, b1, w2t, b2):
    nb = NPAD // BR

    def body(h_ref, p0_ref, p1_ref, w1_ref, b1_ref, w2_ref, b2_ref, z_ref, st_ref):
        i = pl.program_id(0)

        @pl.when(i == 0)
        def _():
            st_ref[...] = jnp.zeros_like(st_ref)

        x = h_ref[...] + p0_ref[...] + p1_ref[...]
        z1 = jnp.maximum(
            jnp.dot(x, w1_ref[...], preferred_element_type=jnp.float32)
            + b1_ref[...], 0.0)
        z = jnp.maximum(
            jnp.dot(z1, w2_ref[...], preferred_element_type=jnp.float32)
            + b2_ref[...], 0.0)
        rid = i * BR + lax.broadcasted_iota(jnp.int32, (BR, 1), 0)
        z = jnp.where(rid < N, z, 0.0)
        z_ref[...] = z
        st_ref[0:1, :] += jnp.sum(z, axis=0, keepdims=True)
        st_ref[1:2, :] += jnp.sum(z * z, axis=0, keepdims=True)

    return pl.pallas_call(
        body,
        grid=(nb,),
        in_specs=[
            pl.BlockSpec((BR, D), lambda i: (i, 0)),
            pl.BlockSpec((BR, D), lambda i: (i, 0)),
            pl.BlockSpec((BR, D), lambda i: (i, 0)),
            pl.BlockSpec((D, D), lambda i: (0, 0)),
            pl.BlockSpec((1, D), lambda i: (0, 0)),
            pl.BlockSpec((D, D), lambda i: (0, 0)),
            pl.BlockSpec((1, D), lambda i: (0, 0)),
        ],
        out_specs=[
            pl.BlockSpec((BR, D), lambda i: (i, 0)),
            pl.BlockSpec((8, D), lambda i: (0, 0)),
        ],
        out_shape=[
            jax.ShapeDtypeStruct((NPAD, D), jnp.float32),
            jax.ShapeDtypeStruct((8, D), jnp.float32),
        ],
    )(h, p0, p1, w1t, b1, w2t, b2)


def _bn_apply(z, st, g, bb):
    nb = NPAD // BR

    def body(z_ref, st_ref, g_ref, b_ref, h_ref):
        mu = st_ref[0:1, :] / N
        var = st_ref[1:2, :] / N - mu * mu
        a = g_ref[...] * lax.rsqrt(var + 1e-5)
        c = b_ref[...] - mu * a
        h_ref[...] = z_ref[...] * a + c

    return pl.pallas_call(
        body,
        grid=(nb,),
        in_specs=[
            pl.BlockSpec((BR, D), lambda i: (i, 0)),
            pl.BlockSpec((8, D), lambda i: (0, 0)),
            pl.BlockSpec((1, D), lambda i: (0, 0)),
            pl.BlockSpec((1, D), lambda i: (0, 0)),
        ],
        out_specs=pl.BlockSpec((BR, D), lambda i: (i, 0)),
        out_shape=jax.ShapeDtypeStruct((NPAD, D), jnp.float32),
    )(z, st, g, bb)


def _attention(h, batch_pad, target3, table_pad):
    scale = 1.0 / math.sqrt(float(D))

    def body(batch_ref, tgt_ref, table_ref, h_ref, pooled_ref):
        b = pl.program_id(0)
        bat = batch_ref[...]
        off0 = jnp.sum((bat < b).astype(jnp.int32))
        off1 = jnp.sum((bat <= b).astype(jnp.int32))
        tgt_row = tgt_ref[0]
        oht = (lax.broadcasted_iota(jnp.int32, (32, L), 0) == tgt_row
               ).astype(jnp.float32)
        emb_b = lax.dot_general(oht, table_ref[...],
                                (((0,), (0,)), ((), ())),
                                preferred_element_type=jnp.float32)
        nblk = (off1 - off0 + TN - 1) // TN

        def blk(j, acc):
            start = off0 + j * TN
            rows = h_ref[pl.ds(start, TN), :]
            s = lax.dot_general(rows, emb_b, (((1,), (1,)), ((), ())),
                                preferred_element_type=jnp.float32) * scale
            m = jnp.max(s, axis=1, keepdims=True)
            e = jnp.exp(s - m)
            p = e / jnp.sum(e, axis=1, keepdims=True)
            o = jnp.dot(p, emb_b, preferred_element_type=jnp.float32)
            rid = start + lax.broadcasted_iota(jnp.int32, (TN, 1), 0)
            o = jnp.where(rid < off1, o, 0.0)
            return acc + jnp.sum(o, axis=0, keepdims=True)

        pooled_ref[0] = lax.fori_loop(
            0, nblk, blk, jnp.zeros((1, D), jnp.float32))

    return pl.pallas_call(
        body,
        grid=(B,),
        in_specs=[
            pl.BlockSpec((NPAD // D, D), lambda b: (0, 0)),
            pl.BlockSpec((1, 1, L), lambda b: (b, 0, 0)),
            pl.BlockSpec((32, D), lambda b: (0, 0)),
            pl.BlockSpec((NPAD, D), lambda b: (0, 0)),
        ],
        out_specs=pl.BlockSpec((1, 1, D), lambda b: (b, 0, 0)),
        out_shape=jax.ShapeDtypeStruct((B, 1, D), jnp.float32),
    )(batch_pad, target3, table_pad, h)


def _xt_matmul(targett, table_pad, w, bias):
    kl = 8
    nk = L // kl

    def body(t_ref, table_ref, w_ref, b_ref, o_ref):
        k = pl.program_id(0)

        @pl.when(k == 0)
        def _():
            o_ref[...] = jnp.broadcast_to(b_ref[...], (B, D))

        acc = jnp.zeros((B, D), jnp.float32)
        for j in range(kl):
            qb = lax.dot_general(table_ref[...], w_ref[:, j * D:(j + 1) * D],
                                 (((1,), (1,)), ((), ())),
                                 preferred_element_type=jnp.float32)
            oht = (lax.broadcasted_iota(jnp.int32, (32, B), 0)
                   == t_ref[j:j + 1, :]).astype(jnp.float32)
            acc += lax.dot_general(oht, qb, (((0,), (0,)), ((), ())),
                                   preferred_element_type=jnp.float32)
        o_ref[...] += acc

    return pl.pallas_call(
        body,
        grid=(nk,),
        in_specs=[
            pl.BlockSpec((kl, B), lambda k: (k, 0)),
            pl.BlockSpec((32, D), lambda k: (0, 0)),
            pl.BlockSpec((D, kl * D), lambda k: (0, k)),
            pl.BlockSpec((1, D), lambda k: (0, 0)),
        ],
        out_specs=pl.BlockSpec((B, D), lambda k: (0, 0)),
        out_shape=jax.ShapeDtypeStruct((B, D), jnp.float32),
    )(targett, table_pad, w, bias)


def _head(pooled, xt, fxd_t, fxd_b, fc1_t, fc1_b, fc2_t, fc2_b, outw_p, outb):
    def body(p_ref, xt_ref, w0_ref, b0_ref, w1_ref, b1_ref, w2_ref, b2_ref,
             w3_ref, b3_ref, o_ref):
        xg = jnp.maximum(
            jnp.dot(p_ref[...], w0_ref[...], preferred_element_type=jnp.float32)
            + b0_ref[...], 0.0)
        xc = jnp.concatenate([xg, xt_ref[...]], axis=1)
        h1 = jnp.maximum(
            jnp.dot(xc, w1_ref[...], preferred_element_type=jnp.float32)
            + b1_ref[...], 0.0)
        h2 = jnp.maximum(
            jnp.dot(h1, w2_ref[...], preferred_element_type=jnp.float32)
            + b2_ref[...], 0.0)
        o_ref[...] = (jnp.dot(h2, w3_ref[...], preferred_element_type=jnp.float32)
                      + b3_ref[...])

    return pl.pallas_call(
        body,
        out_shape=jax.ShapeDtypeStruct((B, D), jnp.float32),
    )(pooled, xt, fxd_t, fxd_b, fc1_t, fc1_b, fc2_t, fc2_b, outw_p, outb)


def kernel(x, edge_index, batch, target, params):
    pad_dst = N + jnp.arange(EPAD - E, dtype=jnp.int32) % (NPAD - N)
    pad_src = jnp.arange(EPAD - E, dtype=jnp.int32) % N
    src2d = jnp.concatenate(
        [edge_index[0], pad_src]).reshape(NCHUNK, CH)
    dst2d = jnp.concatenate(
        [edge_index[1], pad_dst]).reshape(NCHUNK, CH)
    zeros = jnp.zeros((NPAD, D), jnp.float32)
    table_pad = jnp.pad(params['emb'], ((0, 32 - 26), (0, 0)))
    xt = _xt_matmul(target.T, table_pad, params['fc1xt_W'],
                    params['fc1xt_b'].reshape(1, D))
    h = jnp.pad(x, ((0, NPAD - N), (0, D - DIN)))
    for c in (1, 2, 3):
        w1 = params['conv%d_W1' % c]
        if c == 1:
            w1 = jnp.pad(w1, ((0, 0), (0, D - DIN)))
        parts = _seg_partials(h, src2d, dst2d, zeros)
        z, st = _conv_mlp(h, parts[0], parts[1], w1.T,
                          params['conv%d_b1' % c].reshape(1, D),
                          params['conv%d_W2' % c].T,
                          params['conv%d_b2' % c].reshape(1, D))
        h = _bn_apply(z, st, params['bn%d_g' % c].reshape(1, D),
                      params['bn%d_b' % c].reshape(1, D))

    batch_pad = jnp.concatenate(
        [batch, jnp.full((NPAD - N,), B, jnp.int32)]).reshape(NPAD // D, D)
    pooled = _attention(h, batch_pad, target.reshape(B, 1, L), table_pad)
    pooled = pooled.reshape(B, D)

    outw_p = jnp.pad(params['out_W'].T, ((0, 0), (0, D - 1)))
    out = _head(pooled, xt,
                params['fc1xd_W'].T, params['fc1xd_b'].reshape(1, D),
                params['fc1_W'].T, params['fc1_b'].reshape(1, 1024),
                params['fc2_W'].T, params['fc2_b'].reshape(1, 256),
                outw_p, jnp.pad(params['out_b'].reshape(1, 1),
                                ((0, 0), (0, D - 1))))
    return out[:, :1]


# speedup vs baseline: 1.1668x; 1.0479x over previous
import functools
import math

import jax
import jax.numpy as jnp
from jax import lax
from jax.experimental import pallas as pl
from jax.experimental.pallas import tpu as pltpu
from jax.experimental.pallas import tpu_sc as plsc

N, E, B, L, DIN, D = 10000, 320000, 256, 1000, 78, 128
NPAD = 10240
NC, NS = 2, 16
CH = 64
EPAD = 327680
NCHUNK = EPAD // CH
NITER = NCHUNK // (NC * NS)
GS = 32
BR = 1024
TN = 128
LD = L * D


def _seg_partials(h, src2d, dst2d, zeros):
    mesh = plsc.VectorSubcoreMesh(core_axis_name="c", subcore_axis_name="s")
    NBUF = 4

    @functools.partial(
        pl.kernel,
        out_type=jax.ShapeDtypeStruct((NC * NPAD, D), jnp.float32),
        mesh=mesh,
        scratch_types=[
            pltpu.VMEM((GS, CH), jnp.int32),
            pltpu.VMEM((GS, CH), jnp.int32),
        ] + [pltpu.VMEM((CH, D), jnp.float32) for _ in range(NBUF)] + [
            pltpu.VMEM_SHARED((NPAD, D), jnp.float32),
        ] + [pltpu.SemaphoreType.DMA for _ in range(NBUF)],
    )
    def k(h_hbm, src_hbm, dst_hbm, zeros_hbm, out_hbm,
          sidx, didx, *rest):
        rows = rest[:NBUF]
        accum = rest[NBUF]
        sems = rest[NBUF + 1:]
        c = lax.axis_index("c")
        s = lax.axis_index("s")
        rps = NPAD // NS
        pltpu.sync_copy(zeros_hbm.at[pl.ds(s * rps, rps)],
                        accum.at[pl.ds(s * rps, rps)])
        cb = (c * NS + s) * NITER
        plsc.subcore_barrier()

        def group(g, carry):
            pltpu.sync_copy(src_hbm.at[pl.ds(cb + g * GS, GS)], sidx)
            pltpu.sync_copy(dst_hbm.at[pl.ds(cb + g * GS, GS)], didx)
            for j in range(NBUF):
                pltpu.async_copy(h_hbm.at[sidx.at[j]], rows[j], sems[j])

            def body(t, carry2):
                for j in range(NBUF):
                    i = NBUF * t + j
                    pltpu.make_async_copy(
                        h_hbm.at[sidx.at[i]], rows[j], sems[j]).wait()
                    pltpu.sync_copy(rows[j], accum.at[didx.at[i]], add=True)

                    @pl.when(i + NBUF < GS)
                    def _():
                        pltpu.async_copy(
                            h_hbm.at[sidx.at[i + NBUF]], rows[j], sems[j])
                return carry2

            lax.fori_loop(0, GS // NBUF, body, 0)
            return carry

        lax.fori_loop(0, NITER // GS, group, 0)
        plsc.subcore_barrier()
        pltpu.sync_copy(accum.at[pl.ds(s * rps, rps)],
                        out_hbm.at[pl.ds(c * NPAD + s * rps, rps)])

    out = k(h, src2d, dst2d, zeros)
    return out.reshape(NC, NPAD, D)


def _conv_mlp(h, p0, p1, w1t, b1, w2t, b2):
    nb = NPAD // BR

    def body(h_ref, p0_ref, p1_ref, w1_ref, b1_ref, w2_ref, b2_ref, z_ref, st_ref):
        i = pl.program_id(0)

        @pl.when(i == 0)
        def _():
            st_ref[...] = jnp.zeros_like(st_ref)

        x = h_ref[...] + p0_ref[...] + p1_ref[...]
        z1 = jnp.maximum(
            jnp.dot(x, w1_ref[...], preferred_element_type=jnp.float32)
            + b1_ref[...], 0.0)
        z = jnp.maximum(
            jnp.dot(z1, w2_ref[...], preferred_element_type=jnp.float32)
            + b2_ref[...], 0.0)
        rid = i * BR + lax.broadcasted_iota(jnp.int32, (BR, 1), 0)
        z = jnp.where(rid < N, z, 0.0)
        z_ref[...] = z
        st_ref[0:1, :] += jnp.sum(z, axis=0, keepdims=True)
        st_ref[1:2, :] += jnp.sum(z * z, axis=0, keepdims=True)

    return pl.pallas_call(
        body,
        grid=(nb,),
        in_specs=[
            pl.BlockSpec((BR, D), lambda i: (i, 0)),
            pl.BlockSpec((BR, D), lambda i: (i, 0)),
            pl.BlockSpec((BR, D), lambda i: (i, 0)),
            pl.BlockSpec((D, D), lambda i: (0, 0)),
            pl.BlockSpec((1, D), lambda i: (0, 0)),
            pl.BlockSpec((D, D), lambda i: (0, 0)),
            pl.BlockSpec((1, D), lambda i: (0, 0)),
        ],
        out_specs=[
            pl.BlockSpec((BR, D), lambda i: (i, 0)),
            pl.BlockSpec((8, D), lambda i: (0, 0)),
        ],
        out_shape=[
            jax.ShapeDtypeStruct((NPAD, D), jnp.float32),
            jax.ShapeDtypeStruct((8, D), jnp.float32),
        ],
    )(h, p0, p1, w1t, b1, w2t, b2)


def _bn_apply(z, st, g, bb):
    nb = NPAD // BR

    def body(z_ref, st_ref, g_ref, b_ref, h_ref):
        mu = st_ref[0:1, :] / N
        var = st_ref[1:2, :] / N - mu * mu
        a = g_ref[...] * lax.rsqrt(var + 1e-5)
        c = b_ref[...] - mu * a
        h_ref[...] = z_ref[...] * a + c

    return pl.pallas_call(
        body,
        grid=(nb,),
        in_specs=[
            pl.BlockSpec((BR, D), lambda i: (i, 0)),
            pl.BlockSpec((8, D), lambda i: (0, 0)),
            pl.BlockSpec((1, D), lambda i: (0, 0)),
            pl.BlockSpec((1, D), lambda i: (0, 0)),
        ],
        out_specs=pl.BlockSpec((BR, D), lambda i: (i, 0)),
        out_shape=jax.ShapeDtypeStruct((NPAD, D), jnp.float32),
    )(z, st, g, bb)


def _ht(h, tablet):
    nb = NPAD // BR
    scale = 1.0 / math.sqrt(float(D))

    def body(h_ref, t_ref, o_ref):
        o_ref[...] = jnp.dot(h_ref[...], t_ref[...],
                             preferred_element_type=jnp.float32) * scale

    return pl.pallas_call(
        body,
        grid=(nb,),
        in_specs=[
            pl.BlockSpec((BR, D), lambda i: (i, 0)),
            pl.BlockSpec((D, 32), lambda i: (0, 0)),
        ],
        out_specs=pl.BlockSpec((BR, 32), lambda i: (i, 0)),
        out_shape=jax.ShapeDtypeStruct((NPAD, 32), jnp.float32),
    )(h, tablet)


def _attention(ht, batch_pad, target3, table_pad):
    def body(batch_ref, tgt_ref, table_ref, ht_ref, pooled_ref):
        b = pl.program_id(0)
        bat = batch_ref[...]
        off0 = jnp.sum((bat < b).astype(jnp.int32))
        off1 = jnp.sum((bat <= b).astype(jnp.int32))
        tgt_row = tgt_ref[0]
        oht = (lax.broadcasted_iota(jnp.int32, (32, L), 0) == tgt_row
               ).astype(jnp.float32)
        nblk = (off1 - off0 + TN - 1) // TN

        def blk(j, acc):
            start = off0 + j * TN
            rows = ht_ref[pl.ds(start, TN), :]
            s = jnp.dot(rows, oht, preferred_element_type=jnp.float32)
            m = jnp.max(s, axis=1, keepdims=True)
            e = jnp.exp(s - m)
            p = e / jnp.sum(e, axis=1, keepdims=True)
            c = lax.dot_general(p, oht, (((1,), (1,)), ((), ())),
                                preferred_element_type=jnp.float32)
            rid = start + lax.broadcasted_iota(jnp.int32, (TN, 1), 0)
            c = jnp.where(rid < off1, c, 0.0)
            return acc + jnp.sum(c, axis=0, keepdims=True)

        csum = lax.fori_loop(0, nblk, blk, jnp.zeros((1, 32), jnp.float32))
        pooled_ref[0] = jnp.dot(csum, table_ref[...],
                                preferred_element_type=jnp.float32)

    return pl.pallas_call(
        body,
        grid=(B,),
        in_specs=[
            pl.BlockSpec((NPAD // D, D), lambda b: (0, 0)),
            pl.BlockSpec((1, 1, L), lambda b: (b, 0, 0)),
            pl.BlockSpec((32, D), lambda b: (0, 0)),
            pl.BlockSpec((NPAD, 32), lambda b: (0, 0)),
        ],
        out_specs=pl.BlockSpec((1, 1, D), lambda b: (b, 0, 0)),
        out_shape=jax.ShapeDtypeStruct((B, 1, D), jnp.float32),
    )(batch_pad, target3, table_pad, ht)


def _xt_matmul(targett, table_pad, w, bias):
    kl = 8
    nk = L // kl

    def body(t_ref, table_ref, w_ref, b_ref, o_ref):
        k = pl.program_id(0)

        @pl.when(k == 0)
        def _():
            o_ref[...] = jnp.broadcast_to(b_ref[...], (B, D))

        acc = jnp.zeros((B, D), jnp.float32)
        for j in range(kl):
            qb = lax.dot_general(table_ref[...], w_ref[:, j * D:(j + 1) * D],
                                 (((1,), (1,)), ((), ())),
                                 preferred_element_type=jnp.float32)
            oht = (lax.broadcasted_iota(jnp.int32, (32, B), 0)
                   == t_ref[j:j + 1, :]).astype(jnp.float32)
            acc += lax.dot_general(oht, qb, (((0,), (0,)), ((), ())),
                                   preferred_element_type=jnp.float32)
        o_ref[...] += acc

    return pl.pallas_call(
        body,
        grid=(nk,),
        in_specs=[
            pl.BlockSpec((kl, B), lambda k: (k, 0)),
            pl.BlockSpec((32, D), lambda k: (0, 0)),
            pl.BlockSpec((D, kl * D), lambda k: (0, k)),
            pl.BlockSpec((1, D), lambda k: (0, 0)),
        ],
        out_specs=pl.BlockSpec((B, D), lambda k: (0, 0)),
        out_shape=jax.ShapeDtypeStruct((B, D), jnp.float32),
    )(targett, table_pad, w, bias)


def _head(pooled, xt, fxd_t, fxd_b, fc1_t, fc1_b, fc2_t, fc2_b, outw_p, outb):
    def body(p_ref, xt_ref, w0_ref, b0_ref, w1_ref, b1_ref, w2_ref, b2_ref,
             w3_ref, b3_ref, o_ref):
        xg = jnp.maximum(
            jnp.dot(p_ref[...], w0_ref[...], preferred_element_type=jnp.float32)
            + b0_ref[...], 0.0)
        xc = jnp.concatenate([xg, xt_ref[...]], axis=1)
        h1 = jnp.maximum(
            jnp.dot(xc, w1_ref[...], preferred_element_type=jnp.float32)
            + b1_ref[...], 0.0)
        h2 = jnp.maximum(
            jnp.dot(h1, w2_ref[...], preferred_element_type=jnp.float32)
            + b2_ref[...], 0.0)
        o_ref[...] = (jnp.dot(h2, w3_ref[...], preferred_element_type=jnp.float32)
                      + b3_ref[...])

    return pl.pallas_call(
        body,
        out_shape=jax.ShapeDtypeStruct((B, D), jnp.float32),
    )(pooled, xt, fxd_t, fxd_b, fc1_t, fc1_b, fc2_t, fc2_b, outw_p, outb)


def kernel(x, edge_index, batch, target, params):
    pad_dst = N + jnp.arange(EPAD - E, dtype=jnp.int32) % (NPAD - N)
    pad_src = jnp.arange(EPAD - E, dtype=jnp.int32) % N
    src2d = jnp.concatenate(
        [edge_index[0], pad_src]).reshape(NCHUNK, CH)
    dst2d = jnp.concatenate(
        [edge_index[1], pad_dst]).reshape(NCHUNK, CH)
    zeros = jnp.zeros((NPAD, D), jnp.float32)
    table_pad = jnp.pad(params['emb'], ((0, 32 - 26), (0, 0)))
    xt = _xt_matmul(target.T, table_pad, params['fc1xt_W'],
                    params['fc1xt_b'].reshape(1, D))
    h = jnp.pad(x, ((0, NPAD - N), (0, D - DIN)))
    for c in (1, 2, 3):
        w1 = params['conv%d_W1' % c]
        if c == 1:
            w1 = jnp.pad(w1, ((0, 0), (0, D - DIN)))
        parts = _seg_partials(h, src2d, dst2d, zeros)
        z, st = _conv_mlp(h, parts[0], parts[1], w1.T,
                          params['conv%d_b1' % c].reshape(1, D),
                          params['conv%d_W2' % c].T,
                          params['conv%d_b2' % c].reshape(1, D))
        h = _bn_apply(z, st, params['bn%d_g' % c].reshape(1, D),
                      params['bn%d_b' % c].reshape(1, D))

    batch_pad = jnp.concatenate(
        [batch, jnp.full((NPAD - N,), B, jnp.int32)]).reshape(NPAD // D, D)
    ht = _ht(h, table_pad.T)
    pooled = _attention(ht, batch_pad, target.reshape(B, 1, L), table_pad)
    pooled = pooled.reshape(B, D)

    outw_p = jnp.pad(params['out_W'].T, ((0, 0), (0, D - 1)))
    out = _head(pooled, xt,
                params['fc1xd_W'].T, params['fc1xd_b'].reshape(1, D),
                params['fc1_W'].T, params['fc1_b'].reshape(1, 1024),
                params['fc2_W'].T, params['fc2_b'].reshape(1, 256),
                outw_p, jnp.pad(params['out_b'].reshape(1, 1),
                                ((0, 0), (0, D - 1))))
    return out[:, :1]
